# baseline (speedup 1.0000x reference)
import functools

import jax
import jax.numpy as jnp
from jax.experimental import pallas as pl
from jax.experimental.pallas import tpu as pltpu


def _fused_kernel(x_ref, rw_ref, rb_ref, ew_ref, eb_ref, out_ref, aux_ref,
                  scale_ref, eidx_ref, *, T, E, capacity, alpha, q_chunk):
    e = pl.program_id(0)
    x = x_ref[...]

    @pl.when(e == 0)
    def _router():
        logits = jax.lax.dot_general(
            x, rw_ref[...], (((1,), (1,)), ((), ())),
            preferred_element_type=jnp.float32) + rb_ref[0:1, :]
        m = jnp.max(logits, axis=1, keepdims=True)
        ex = jnp.exp(logits - m)
        probs = ex / jnp.sum(ex, axis=1, keepdims=True)

        p = jnp.max(probs, axis=1, keepdims=True)
        e_iota = jax.lax.broadcasted_iota(jnp.int32, (T, E), 1)
        eidx = jnp.min(jnp.where(probs == p, e_iota, E), axis=1,
                       keepdims=True)
        eidx_ref[...] = eidx

        one_hot_p = jnp.where(e_iota == eidx, p, 0.0)
        f_sum = jnp.sum(one_hot_p, axis=0, keepdims=True)
        p_sum = jnp.sum(probs, axis=0, keepdims=True)
        aux_ref[...] = (alpha * E / (T * T)) * jnp.sum(f_sum * p_sum,
                                                       keepdims=True)

        scale_ref[...] = p
        overflow = jnp.max(f_sum) > capacity

        @pl.when(overflow)
        def _capacity():
            p_row = jnp.transpose(p)
            e_row = jnp.transpose(eidx)
            k_idx = jax.lax.broadcasted_iota(jnp.int32, (1, T), 1)
            for c0 in range(0, T, q_chunk):
                pq = p[c0:c0 + q_chunk]
                eq = eidx[c0:c0 + q_chunk]
                qi = jax.lax.broadcasted_iota(
                    jnp.int32, (q_chunk, 1), 0) + c0
                before = (p_row > pq) | ((p_row == pq) & (k_idx <= qi))
                mass = jnp.where(before & (e_row == eq), p_row, 0.0)
                s = jnp.sum(mass, axis=1, keepdims=True)
                keep = (s <= capacity).astype(jnp.float32)
                scale_ref[c0:c0 + q_chunk, :] = keep * pq

    w = ew_ref[0]
    y = jax.lax.dot_general(x, w, (((1,), (1,)), ((), ())),
                            preferred_element_type=jnp.float32)
    y = y + eb_ref[0]
    m = jnp.where(eidx_ref[...] == e, scale_ref[...], 0.0)
    contrib = m * y

    @pl.when(e == 0)
    def _():
        out_ref[...] = contrib

    @pl.when(e != 0)
    def _():
        out_ref[...] += contrib


def kernel(x, router_w, router_b, expert_w, expert_b):
    B, S, D = x.shape
    E = router_w.shape[0]
    T = B * S
    capacity = float(int(T / E * 1.0))
    alpha = 0.01

    xf = x.reshape(T, D)
    rb2 = router_b.reshape(1, E)

    out, aux = pl.pallas_call(
        functools.partial(_fused_kernel, T=T, E=E, capacity=capacity,
                          alpha=alpha, q_chunk=512),
        grid=(E,),
        in_specs=[
            pl.BlockSpec((T, D), lambda e: (0, 0)),
            pl.BlockSpec((E, D), lambda e: (0, 0)),
            pl.BlockSpec((1, E), lambda e: (0, 0)),
            pl.BlockSpec((1, D, D), lambda e: (e, 0, 0)),
            pl.BlockSpec((1, 1, D), lambda e: (e, 0, 0)),
        ],
        out_specs=[
            pl.BlockSpec((T, D), lambda e: (0, 0)),
            pl.BlockSpec((1, 1), lambda e: (0, 0)),
        ],
        out_shape=[
            jax.ShapeDtypeStruct((T, D), jnp.float32),
            jax.ShapeDtypeStruct((1, 1), jnp.float32),
        ],
        scratch_shapes=[
            pltpu.VMEM((T, 1), jnp.float32),
            pltpu.VMEM((T, 1), jnp.int32),
        ],
    )(xf, router_w, rb2, expert_w, expert_b.reshape(E, 1, D))

    return out.reshape(B, S, D), aux[0, 0]
